# baseline (speedup 1.0000x reference)
import functools

import jax
import jax.numpy as jnp
from jax import lax
from jax.experimental import pallas as pl
from jax.experimental.pallas import tpu as pltpu
from jax.experimental.pallas import tpu_sc as plsc

N = 10000
E = 160000
F = 256
FH = 128
NC = 2
NS = 16
EPW = E // NS
CHUNK = 50
NCHUNK = EPW // CHUNK
HG = 5
GCH = NCHUNK // HG
ND = 5
RPW = 624
TAIL = N - NS * RPW
DW = 16
DEGW = 128


def _mesh():
    return plsc.VectorSubcoreMesh(
        core_axis_name="c", subcore_axis_name="s",
        num_cores=NC, num_subcores=NS)


@functools.lru_cache(maxsize=None)
def _build_deg_kernel():
    @functools.partial(
        pl.kernel,
        out_type=[jax.ShapeDtypeStruct((NS, N), jnp.float32),
                  jax.ShapeDtypeStruct((NS, N), jnp.float32)],
        mesh=_mesh(),
        compiler_params=pltpu.CompilerParams(needs_layout_passes=False),
        scratch_types=[
            pltpu.VMEM((EPW,), jnp.int32),
            pltpu.VMEM((N,), jnp.float32),
        ],
    )
    def deg_kernel(src_hbm, dst_hbm, dego_hbm, degi_hbm, idx_v, hist_v):
        c = lax.axis_index("c")
        s = lax.axis_index("s")

        def run(e_hbm, out_hbm):
            def zero(i, x):
                hist_v[pl.ds(i * 16, 16)] = jnp.zeros((16,), jnp.float32)
                return x
            lax.fori_loop(0, N // 16, zero, 0)
            pltpu.sync_copy(e_hbm.at[s], idx_v)
            ones16 = jnp.ones((16,), jnp.float32)

            def body(i, x):
                idx16 = idx_v[pl.ds(i * 16, 16)]
                plsc.addupdate_scatter(hist_v, [idx16], ones16)
                return x
            lax.fori_loop(0, EPW // 16, body, 0)
            pltpu.sync_copy(hist_v, out_hbm.at[s])

        @pl.when(c == 0)
        def _():
            run(src_hbm, dego_hbm)

        @pl.when(c == 1)
        def _():
            run(dst_hbm, degi_hbm)

    return deg_kernel


@functools.lru_cache(maxsize=None)
def _build_agg_kernel():
    @functools.partial(
        pl.kernel,
        out_type=[jax.ShapeDtypeStruct((N, FH), jnp.float32),
                  jax.ShapeDtypeStruct((N, FH), jnp.float32)],
        mesh=_mesh(),
        scratch_types=[
            pltpu.VMEM((GCH, CHUNK), jnp.int32),
            pltpu.VMEM((GCH, CHUNK), jnp.int32),
        ] + [pltpu.VMEM((CHUNK, FH), jnp.float32)] * ND
          + [pltpu.SemaphoreType.DMA] * ND
          + [pltpu.VMEM_SHARED((N, FH), jnp.float32)],
    )
    def agg_kernel(y0_hbm, y1_hbm, src_hbm, dst_hbm, z_hbm,
                   out0_hbm, out1_hbm, src_v, dst_v, *rest):
        bufs = rest[:ND]
        sems = rest[ND:2 * ND]
        acc_sh = rest[2 * ND]
        c = lax.axis_index("c")
        s = lax.axis_index("s")

        def run(y_hbm, out_hbm):
            base = pl.multiple_of(s * RPW, 8)
            pltpu.sync_copy(z_hbm, acc_sh.at[pl.ds(base, RPW)])

            @pl.when(s == NS - 1)
            def _():
                pltpu.sync_copy(z_hbm.at[pl.ds(0, TAIL)],
                                acc_sh.at[pl.ds(NS * RPW, TAIL)])
            plsc.subcore_barrier()

            for g in range(HG):
                pltpu.sync_copy(src_hbm.at[s, g], src_v)
                pltpu.sync_copy(dst_hbm.at[s, g], dst_v)
                for k in range(ND):
                    pltpu.async_copy(y_hbm.at[src_v.at[k]], bufs[k], sems[k])

                def body(jj, x):
                    j0 = jj * ND

                    def step(k):
                        pltpu.make_async_copy(
                            y_hbm.at[src_v.at[j0 + k]], bufs[k],
                            sems[k]).wait()
                        pltpu.sync_copy(bufs[k], acc_sh.at[dst_v.at[j0 + k]],
                                        add=True)

                        @pl.when(jj < GCH // ND - 1)
                        def _():
                            pltpu.async_copy(
                                y_hbm.at[src_v.at[j0 + ND + k]],
                                bufs[k], sems[k])
                    for k in range(ND):
                        step(k)
                    return x
                lax.fori_loop(0, GCH // ND, body, 0)

            plsc.subcore_barrier()
            pltpu.sync_copy(acc_sh.at[pl.ds(base, RPW)],
                            out_hbm.at[pl.ds(base, RPW)])

            @pl.when(s == NS - 1)
            def _():
                pltpu.sync_copy(acc_sh.at[pl.ds(NS * RPW, TAIL)],
                                out_hbm.at[pl.ds(NS * RPW, TAIL)])

        @pl.when(c == 0)
        def _():
            run(y0_hbm, out0_hbm)

        @pl.when(c == 1)
        def _():
            run(y1_hbm, out1_hbm)

    return agg_kernel


def _norm(deg_col):
    return jnp.where(deg_col > 0.0,
                     lax.rsqrt(jnp.maximum(deg_col, 1e-12)), 0.0)


def _mm0_body(feat_ref, dego_ref, w_ref, y0_ref, y1_ref):
    nsrc = _norm(jnp.sum(dego_ref[...], axis=1, keepdims=True))
    y = jnp.dot(feat_ref[...] * nsrc, w_ref[...],
                preferred_element_type=jnp.float32)
    y0_ref[...] = y[:, :FH]
    y1_ref[...] = y[:, FH:]


def _mm_body(a0_ref, a1_ref, degi_ref, dego_ref, b_ref, w_ref, y0_ref, y1_ref):
    ndst = _norm(jnp.sum(degi_ref[...], axis=1, keepdims=True))
    nsrc = _norm(jnp.sum(dego_ref[...], axis=1, keepdims=True))
    h = jnp.concatenate([a0_ref[...], a1_ref[...]], axis=-1)
    h = jnp.maximum(h * ndst + b_ref[...], 0.0) * nsrc
    y = jnp.dot(h, w_ref[...], preferred_element_type=jnp.float32)
    y0_ref[...] = y[:, :FH]
    y1_ref[...] = y[:, FH:]


def _fin_body(a0_ref, a1_ref, degi_ref, b_ref, out_ref):
    ndst = _norm(jnp.sum(degi_ref[...], axis=1, keepdims=True))
    h = jnp.concatenate([a0_ref[...], a1_ref[...]], axis=-1)
    out_ref[...] = h * ndst + b_ref[...]


_R = 1000


def _mm0(features, dego, w):
    return pl.pallas_call(
        _mm0_body,
        grid=(N // _R,),
        in_specs=[pl.BlockSpec((_R, F), lambda i: (i, 0)),
                  pl.BlockSpec((_R, NS), lambda i: (i, 0)),
                  pl.BlockSpec((F, F), lambda i: (0, 0))],
        out_specs=[pl.BlockSpec((_R, FH), lambda i: (i, 0)),
                   pl.BlockSpec((_R, FH), lambda i: (i, 0))],
        out_shape=[jax.ShapeDtypeStruct((N, FH), jnp.float32)] * 2,
    )(features, dego, w)


def _mm(a0, a1, degi, dego, b, w):
    return pl.pallas_call(
        _mm_body,
        grid=(N // _R,),
        in_specs=[pl.BlockSpec((_R, FH), lambda i: (i, 0)),
                  pl.BlockSpec((_R, FH), lambda i: (i, 0)),
                  pl.BlockSpec((_R, NS), lambda i: (i, 0)),
                  pl.BlockSpec((_R, NS), lambda i: (i, 0)),
                  pl.BlockSpec((1, F), lambda i: (0, 0)),
                  pl.BlockSpec((F, F), lambda i: (0, 0))],
        out_specs=[pl.BlockSpec((_R, FH), lambda i: (i, 0)),
                   pl.BlockSpec((_R, FH), lambda i: (i, 0))],
        out_shape=[jax.ShapeDtypeStruct((N, FH), jnp.float32)] * 2,
    )(a0, a1, degi, dego, b, w)


def _fin(a0, a1, degi, b):
    return pl.pallas_call(
        _fin_body,
        grid=(N // _R,),
        in_specs=[pl.BlockSpec((_R, FH), lambda i: (i, 0)),
                  pl.BlockSpec((_R, FH), lambda i: (i, 0)),
                  pl.BlockSpec((_R, NS), lambda i: (i, 0)),
                  pl.BlockSpec((1, F), lambda i: (0, 0))],
        out_specs=pl.BlockSpec((_R, F), lambda i: (i, 0)),
        out_shape=jax.ShapeDtypeStruct((N, F), jnp.float32),
    )(a0, a1, degi, b)


def kernel(features, edge_index, W0, b0, W1, b1, W2, b2):
    src = edge_index[0].reshape(NS, HG, GCH, CHUNK)
    dst = edge_index[1].reshape(NS, HG, GCH, CHUNK)
    zagg = jnp.zeros((RPW, FH), jnp.float32)
    b0r = b0.reshape(1, F)
    b1r = b1.reshape(1, F)
    b2r = b2.reshape(1, F)

    src_rows = edge_index[0].reshape(NS, EPW)
    dst_rows = edge_index[1].reshape(NS, EPW)
    parto, parti = _build_deg_kernel()(src_rows, dst_rows)
    dego = parto.T
    degi = parti.T
    y0a, y0b = _mm0(features, dego, W0)
    a0a, a0b = _build_agg_kernel()(y0a, y0b, src, dst, zagg)
    y1a, y1b = _mm(a0a, a0b, degi, dego, b0r, W1)
    a1a, a1b = _build_agg_kernel()(y1a, y1b, src, dst, zagg)
    y2a, y2b = _mm(a1a, a1b, degi, dego, b1r, W2)
    a2a, a2b = _build_agg_kernel()(y2a, y2b, src, dst, zagg)
    return _fin(a2a, a2b, degi, b2r)
